# Initial kernel scaffold; baseline (speedup 1.0000x reference)
import jax
import jax.numpy as jnp
from jax.experimental import pallas as pl


def kernel(x, edge_index, W):
    raise NotImplementedError("write your pallas kernel here")


# speedup vs baseline: 15.5335x; 15.5335x over previous
import functools

import jax
import jax.numpy as jnp
from jax import lax
from jax.experimental import pallas as pl
from jax.experimental.pallas import tpu as pltpu
from jax.experimental.pallas import tpu_sc as plsc

N_NODES = 10000
D = 128
T = 4
TAU = 2.0
V_TH = 1.0
DELTA = 0.05
STEP_SIZE = 0.1

NC = 2
NS = 16
NW = NC * NS

N_PAD = 10240
ROWS_PER_SUB = N_PAD // NS

CHUNK = 80
DEG_W = 1


def _sc_mesh():
    return plsc.VectorSubcoreMesh(core_axis_name="c", subcore_axis_name="s")


def _deg_body(dst_hbm, out_hbm, idx_v, ones_v, zbuf_v, acc_sh):
    cid = lax.axis_index("c")
    sid = lax.axis_index("s")
    wid = cid * NS + sid
    e_per_w = dst_hbm.shape[0] // NW
    n_chunks = e_per_w // CHUNK

    for i in range(CHUNK // 16):
        ones_v[pl.ds(i * 16, 16)] = jnp.full((16,), 1.0, jnp.float32)
    for i in range(ROWS_PER_SUB // 16):
        zbuf_v[pl.ds(i * 16, 16)] = jnp.zeros((16,), jnp.float32)

    row0 = sid * ROWS_PER_SUB
    pltpu.sync_copy(zbuf_v, acc_sh.at[pl.ds(row0, ROWS_PER_SUB)])
    plsc.subcore_barrier()

    base = wid * e_per_w

    def body(c, carry):
        pltpu.sync_copy(dst_hbm.at[pl.ds(base + c * CHUNK, CHUNK)], idx_v)
        pltpu.sync_copy(ones_v, acc_sh.at[idx_v], add=True)
        return carry

    lax.fori_loop(0, n_chunks, body, 0)
    plsc.subcore_barrier()

    pltpu.sync_copy(acc_sh.at[pl.ds(row0, ROWS_PER_SUB)],
                    out_hbm.at[pl.ds(cid * N_PAD + row0, ROWS_PER_SUB)])


def _deg_counts(dst):
    kfn = pl.kernel(
        _deg_body,
        out_type=jax.ShapeDtypeStruct((NC * N_PAD,), jnp.float32),
        mesh=_sc_mesh(),
        scratch_types=[
            pltpu.VMEM((CHUNK,), jnp.int32),
            pltpu.VMEM((CHUNK,), jnp.float32),
            pltpu.VMEM((ROWS_PER_SUB,), jnp.float32),
            pltpu.VMEM_SHARED((N_PAD,), jnp.float32),
        ],
    )
    return kfn(dst).reshape(NC, N_PAD)[:, :, None]


def _agg_body(src_hbm, dst_hbm, xs_hbm, zeros_hbm, out_hbm,
              sidx_v, didx_v, rows_v, sem, acc_sh):
    cid = lax.axis_index("c")
    sid = lax.axis_index("s")
    wid = cid * NS + sid
    e_per_w = dst_hbm.shape[0] // NW
    n_chunks = e_per_w // CHUNK

    row0 = sid * ROWS_PER_SUB
    pltpu.sync_copy(zeros_hbm.at[pl.ds(row0, ROWS_PER_SUB), :],
                    acc_sh.at[pl.ds(row0, ROWS_PER_SUB), :])
    plsc.subcore_barrier()

    base = wid * e_per_w

    def body(c, carry):
        off = base + c * CHUNK
        pltpu.sync_copy(src_hbm.at[pl.ds(off, CHUNK)], sidx_v)
        pltpu.sync_copy(dst_hbm.at[pl.ds(off, CHUNK)], didx_v)
        pltpu.async_copy(xs_hbm.at[sidx_v], rows_v, sem).wait()
        pltpu.sync_copy(rows_v, acc_sh.at[didx_v], add=True)
        return carry

    lax.fori_loop(0, n_chunks, body, 0)
    plsc.subcore_barrier()

    pltpu.sync_copy(acc_sh.at[pl.ds(row0, ROWS_PER_SUB), :],
                    out_hbm.at[cid, pl.ds(row0, ROWS_PER_SUB), :])


def _aggregate(src, dst, xs):
    kfn = pl.kernel(
        _agg_body,
        out_type=jax.ShapeDtypeStruct((NC, N_PAD, D), jnp.float32),
        mesh=_sc_mesh(),
        scratch_types=[
            pltpu.VMEM((CHUNK,), jnp.int32),
            pltpu.VMEM((CHUNK,), jnp.int32),
            pltpu.VMEM((CHUNK, D), jnp.float32),
            pltpu.SemaphoreType.DMA,
            pltpu.VMEM_SHARED((N_PAD, D), jnp.float32),
        ],
    )
    zeros = jnp.zeros((N_PAD, D), jnp.float32)
    return kfn(src, dst, xs, zeros)


def _scale_body(x_ref, degp_ref, xs_ref):
    deg = 1.0 + degp_ref[0, :, 0:1] + degp_ref[1, :, 0:1]
    dinv = lax.rsqrt(deg)
    xs_ref[...] = x_ref[...] * dinv


def _scale(x, degp, rows):
    grid = (N_NODES // rows,)
    return pl.pallas_call(
        _scale_body,
        grid=grid,
        in_specs=[
            pl.BlockSpec((rows, D), lambda i: (i, 0)),
            pl.BlockSpec((NC, rows, DEG_W), lambda i: (0, i, 0)),
        ],
        out_specs=pl.BlockSpec((rows, D), lambda i: (i, 0)),
        out_shape=jax.ShapeDtypeStruct((N_NODES, D), jnp.float32),
    )(x, degp)


def _final_body(aggp_ref, x_ref, degp_ref, w_ref, o_ref, z_ref):
    deg = 1.0 + degp_ref[0, :, 0:1] + degp_ref[1, :, 0:1]
    dinv = lax.rsqrt(deg)
    agg = aggp_ref[0] + aggp_ref[1]
    u = dinv * agg + (dinv * dinv) * x_ref[...]
    h = jnp.dot(u, w_ref[...], preferred_element_type=jnp.float32)

    dx = h * STEP_SIZE
    v = jnp.zeros_like(h)
    z = jnp.zeros_like(h)
    for t in range(T):
        v = v + (h - v) * (1.0 / TAU)
        o = (v >= V_TH).astype(jnp.float32)
        v = v - o * (V_TH - DELTA)
        z = z + dx * o
        o_ref[t] = o
        z_ref[t] = z


def _final(aggp, x, degp, W, rows):
    grid = (N_NODES // rows,)
    out_shape = jax.ShapeDtypeStruct((T, N_NODES, D), jnp.float32)
    return pl.pallas_call(
        _final_body,
        grid=grid,
        in_specs=[
            pl.BlockSpec((NC, rows, D), lambda i: (0, i, 0)),
            pl.BlockSpec((rows, D), lambda i: (i, 0)),
            pl.BlockSpec((NC, rows, DEG_W), lambda i: (0, i, 0)),
            pl.BlockSpec((D, D), lambda i: (0, 0)),
        ],
        out_specs=[
            pl.BlockSpec((T, rows, D), lambda i: (0, i, 0)),
            pl.BlockSpec((T, rows, D), lambda i: (0, i, 0)),
        ],
        out_shape=[out_shape, out_shape],
    )(aggp, x, degp, W)


def kernel(x, edge_index, W):
    src = edge_index[0].astype(jnp.int32)
    dst = edge_index[1].astype(jnp.int32)

    degp = _deg_counts(dst)
    xs = _scale(x, degp, rows=1000)
    aggp = _aggregate(src, dst, xs)
    o_seq, z_seq = _final(aggp, x, degp, W, rows=1000)
    return (o_seq, z_seq)
